# baseline (speedup 1.0000x reference)
import functools

import jax
import jax.numpy as jnp
from jax import lax
from jax.experimental import pallas as pl
from jax.experimental.pallas import tpu as pltpu
from jax.experimental.pallas import tpu_sc as plsc

N = 10000
DH = 16
NSC = 2
NTILES = 16
NW = NSC * NTILES
CH = 256
NCH0 = 64
NCH1 = 16
NCHMAX = max(NCH0, NCH1)
NROWS = NTILES * (NCH0 + NCH1) + NCHMAX
EPAD = NROWS * CH
NSINK = N + 112
ZROWS = NSINK // NTILES
NBUF = 4
LOOKAHEAD = 2

_f32 = jnp.float32


def _sc_body(*refs):
    (p_hbm, srcs_hbm, dsts_hbm, z_hbm,
     s_out,
     srcs_v, dsts_v,
     rb0, rb1, rb2, rb3, zbuf, s_sp,
     gs0, gs1, gs2, gs3, ss0, ss1, ss2, ss3) = refs
    rows = (rb0, rb1, rb2, rb3)
    gsem = (gs0, gs1, gs2, gs3)
    ssem = (ss0, ss1, ss2, ss3)

    c = lax.axis_index("c")
    t = lax.axis_index("s")
    nch = jnp.where(c == 0, NCH0, NCH1)
    row0 = jnp.where(c == 0, t * NCH0, NTILES * NCH0 + t * NCH1)

    stripe = pl.ds(t * ZROWS, ZROWS)
    pltpu.sync_copy(z_hbm.at[stripe], zbuf)
    pltpu.sync_copy(zbuf, s_sp.at[stripe])
    pltpu.sync_copy(srcs_hbm.at[pl.ds(row0, NCHMAX)], srcs_v)
    pltpu.sync_copy(dsts_hbm.at[pl.ds(row0, NCHMAX)], dsts_v)
    plsc.subcore_barrier()

    def start_gather(j, b):
        pltpu.async_copy(p_hbm.at[srcs_v.at[j]], rows[b], gsem[b])

    def wait_gather(b):
        pltpu.make_async_copy(p_hbm.at[srcs_v.at[0]], rows[b], gsem[b]).wait()

    def start_scatter(j, b):
        pltpu.async_copy(rows[b], s_sp.at[dsts_v.at[j]], ssem[b], add=True)

    def wait_scatter(b):
        pltpu.make_async_copy(rows[b], s_sp.at[dsts_v.at[0]], ssem[b]).wait()

    start_gather(0, 0)
    start_gather(1, 1)
    for j in (0, 1):
        wait_gather(j % NBUF)
        start_scatter(j, j % NBUF)
        start_gather(j + LOOKAHEAD, (j + LOOKAHEAD) % NBUF)

    def main_body(g, carry):
        for b in range(NBUF):
            j = g * NBUF + b + LOOKAHEAD
            bb = (b + LOOKAHEAD) % NBUF
            wait_gather(bb)
            start_scatter(j, bb)
            wait_scatter(b)
            start_gather(j + LOOKAHEAD, b)
        return carry

    lax.fori_loop(0, (nch - 2 * LOOKAHEAD) // NBUF, main_body, 0)

    for k in range(LOOKAHEAD, 0, -1):
        j = nch - k
        b = (NBUF - k) % NBUF
        wait_gather(b)
        start_scatter(j, b)
    for b in range(NBUF):
        wait_scatter(b)
    plsc.subcore_barrier()

    pltpu.sync_copy(s_sp.at[stripe], zbuf)
    pltpu.sync_copy(zbuf, s_out.at[c, stripe])


def _make_sc(width):
    scratch = [
        pltpu.VMEM((NCHMAX, CH), jnp.int32),
        pltpu.VMEM((NCHMAX, CH), jnp.int32),
    ]
    scratch += [pltpu.VMEM((CH, width), _f32)] * NBUF
    scratch += [
        pltpu.VMEM((ZROWS, width), _f32),
        pltpu.VMEM_SHARED((NSINK, width), _f32),
    ]
    scratch += [pltpu.SemaphoreType.DMA] * (2 * NBUF)
    mesh = plsc.VectorSubcoreMesh(core_axis_name="c", subcore_axis_name="s")
    return pl.kernel(
        _sc_body,
        out_type=jax.ShapeDtypeStruct((NSC, NSINK, width), _f32),
        mesh=mesh,
        scratch_types=scratch,
        compiler_params=pltpu.CompilerParams(use_tc_tiling_on_sc=False,
                                             skip_device_barrier=True),
    )


_BN = 1000


def _elu(h):
    return jnp.where(h > 0, h, jnp.exp(jnp.minimum(h, 0.0)) - 1.0)


def _dense_pre_body(x_ref, wl_ref, wr_ref, bl_ref, p_ref, r_ref):
    xb = x_ref[...]
    pj = jnp.dot(xb, wl_ref[...], preferred_element_type=_f32)
    p_ref[...] = jnp.concatenate(
        [pj, jnp.ones((pj.shape[0], 1), _f32), jnp.zeros((pj.shape[0], 15), _f32)],
        axis=1)
    r_ref[...] = jnp.dot(xb, wr_ref[...], preferred_element_type=_f32) + bl_ref[...]


def _dense_pre(x, Wl, Wr, bl):
    n, d = x.shape
    return pl.pallas_call(
        _dense_pre_body,
        grid=(n // _BN,),
        in_specs=[
            pl.BlockSpec((_BN, d), lambda i: (i, 0)),
            pl.BlockSpec((d, DH), lambda i: (0, 0)),
            pl.BlockSpec((d, DH), lambda i: (0, 0)),
            pl.BlockSpec((1, DH), lambda i: (0, 0)),
        ],
        out_specs=[pl.BlockSpec((_BN, 2 * DH), lambda i: (i, 0)),
                   pl.BlockSpec((_BN, DH), lambda i: (i, 0))],
        out_shape=[jax.ShapeDtypeStruct((n, 2 * DH), _f32),
                   jax.ShapeDtypeStruct((n, DH), _f32)],
    )(x, Wl, Wr, bl.reshape(1, DH))


def _combine(s_ref, cnt_ref, r_ref):
    ssum = s_ref[0, :, :DH] + s_ref[1, :, :DH]
    cnt = cnt_ref[0, :, DH:DH + 1] + cnt_ref[1, :, DH:DH + 1]
    inv = 1.0 / jnp.maximum(cnt, 1.0)
    return _elu(ssum * inv + r_ref[...])


def _combine_pre_body(s_ref, cnt_ref, r_ref, wl_ref, wr_ref, bl_ref,
                      p_ref, rout_ref):
    h = _combine(s_ref, cnt_ref, r_ref)
    p_ref[...] = jnp.dot(h, wl_ref[...], preferred_element_type=_f32)
    rout_ref[...] = jnp.dot(h, wr_ref[...], preferred_element_type=_f32) + bl_ref[...]


def _combine_pre(s, cnt, r, Wl, Wr, bl):
    sw = s.shape[-1]
    return pl.pallas_call(
        _combine_pre_body,
        grid=(N // _BN,),
        in_specs=[
            pl.BlockSpec((NSC, _BN, sw), lambda i: (0, i, 0)),
            pl.BlockSpec((NSC, _BN, 2 * DH), lambda i: (0, i, 0)),
            pl.BlockSpec((_BN, DH), lambda i: (i, 0)),
            pl.BlockSpec((DH, DH), lambda i: (0, 0)),
            pl.BlockSpec((DH, DH), lambda i: (0, 0)),
            pl.BlockSpec((1, DH), lambda i: (0, 0)),
        ],
        out_specs=[pl.BlockSpec((_BN, DH), lambda i: (i, 0))] * 2,
        out_shape=[jax.ShapeDtypeStruct((N, DH), _f32)] * 2,
    )(s, cnt, r, Wl, Wr, bl.reshape(1, DH))


def _combine_mlp_body(s_ref, cnt_ref, r_ref, w0, b0, w1, b1, w2, b2, w3, b3,
                      out_ref):
    h = _combine(s_ref, cnt_ref, r_ref)
    h = _elu(jnp.dot(h, w0[...], preferred_element_type=_f32) + b0[...])
    h = _elu(jnp.dot(h, w1[...], preferred_element_type=_f32) + b1[...])
    h = _elu(jnp.dot(h, w2[...], preferred_element_type=_f32) + b2[...])
    out_ref[...] = jnp.dot(h, w3[...], preferred_element_type=_f32) + b3[...]


def _combine_mlp(s, cnt, r, lws):
    (w0, b0), (w1, b1), (w2, b2), (w3, b3) = lws
    d_out = w3.shape[1]
    wspecs = []
    for w, b in lws:
        wspecs.append(pl.BlockSpec(w.shape, lambda i: (0, 0)))
        wspecs.append(pl.BlockSpec((1, b.shape[0]), lambda i: (0, 0)))
    return pl.pallas_call(
        _combine_mlp_body,
        grid=(N // _BN,),
        in_specs=[
            pl.BlockSpec((NSC, _BN, DH), lambda i: (0, i, 0)),
            pl.BlockSpec((NSC, _BN, 2 * DH), lambda i: (0, i, 0)),
            pl.BlockSpec((_BN, DH), lambda i: (i, 0)),
        ] + wspecs,
        out_specs=pl.BlockSpec((_BN, d_out), lambda i: (i, 0)),
        out_shape=jax.ShapeDtypeStruct((N, d_out), _f32),
    )(s, cnt, r, w0, b0.reshape(1, -1), w1, b1.reshape(1, -1),
      w2, b2.reshape(1, -1), w3, b3.reshape(1, -1))


def kernel(x, edge_index,
           conv0_Wl, conv0_bl, conv0_Wr,
           conv1_Wl, conv1_bl, conv1_Wr,
           conv2_Wl, conv2_bl, conv2_Wr,
           lin0_W, lin0_b, lin1_W, lin1_b, lin2_W, lin2_b, lin3_W, lin3_b):
    src = edge_index[0]
    dst = edge_index[1]
    e = src.shape[0]
    pad = EPAD - e
    srcs = jnp.concatenate([src, jnp.zeros((pad,), jnp.int32)]).reshape(NROWS, CH)
    dsts = jnp.concatenate([dst, jnp.full((pad,), N, jnp.int32)]).reshape(NROWS, CH)
    z32 = jnp.zeros((NSINK, 2 * DH), _f32)
    z16 = jnp.zeros((NSINK, DH), _f32)

    sc32 = _make_sc(2 * DH)
    sc16 = _make_sc(DH)

    p0, r0 = _dense_pre(x, conv0_Wl, conv0_Wr, conv0_bl)
    s0p = sc32(p0, srcs, dsts, z32)
    p1, r1 = _combine_pre(s0p, s0p, r0, conv1_Wl, conv1_Wr, conv1_bl)
    s1p = sc16(p1, srcs, dsts, z16)
    p2, r2 = _combine_pre(s1p, s0p, r1, conv2_Wl, conv2_Wr, conv2_bl)
    s2p = sc16(p2, srcs, dsts, z16)
    return _combine_mlp(s2p, s0p, r2,
                        [(lin0_W, lin0_b), (lin1_W, lin1_b),
                         (lin2_W, lin2_b), (lin3_W, lin3_b)])


# speedup vs baseline: 1.0022x; 1.0022x over previous
import functools

import jax
import jax.numpy as jnp
from jax import lax
from jax.experimental import pallas as pl
from jax.experimental.pallas import tpu as pltpu
from jax.experimental.pallas import tpu_sc as plsc

N = 10000
DH = 16
NSC = 2
NTILES = 16
NW = NSC * NTILES
CH = 256
NCH0 = 64
NCH1 = 16
NCHMAX = max(NCH0, NCH1)
NROWS = NTILES * (NCH0 + NCH1) + NCHMAX
EPAD = NROWS * CH
NSINK = N + 112
ZROWS = NSINK // NTILES
NBUF = 4
LOOKAHEAD = 2

_f32 = jnp.float32


def _sc_body(*refs):
    (p_hbm, srcs_hbm, dsts_hbm,
     s_out,
     srcs_v, dsts_v,
     rb0, rb1, rb2, rb3, zbuf, s_sp,
     gs0, gs1, gs2, gs3, ss0, ss1, ss2, ss3) = refs
    rows = (rb0, rb1, rb2, rb3)
    gsem = (gs0, gs1, gs2, gs3)
    ssem = (ss0, ss1, ss2, ss3)

    c = lax.axis_index("c")
    t = lax.axis_index("s")
    nch = jnp.where(c == 0, NCH0, NCH1)
    row0 = jnp.where(c == 0, t * NCH0, NTILES * NCH0 + t * NCH1)

    stripe = pl.ds(t * ZROWS, ZROWS)
    width = zbuf.shape[1]
    zv = jnp.zeros((16,), _f32)

    def zrow(i, carry):
        for h in range(width // 16):
            zbuf[i, pl.ds(h * 16, 16)] = zv
        return carry

    lax.fori_loop(0, ZROWS, zrow, 0)
    pltpu.sync_copy(zbuf, s_sp.at[stripe])
    pltpu.sync_copy(srcs_hbm.at[pl.ds(row0, NCH1)], srcs_v.at[pl.ds(0, NCH1)])
    pltpu.sync_copy(dsts_hbm.at[pl.ds(row0, NCH1)], dsts_v.at[pl.ds(0, NCH1)])

    @pl.when(c == 0)
    def _copy_extra():
        extra = pl.ds(row0 + NCH1, NCH0 - NCH1)
        pltpu.sync_copy(srcs_hbm.at[extra], srcs_v.at[pl.ds(NCH1, NCH0 - NCH1)])
        pltpu.sync_copy(dsts_hbm.at[extra], dsts_v.at[pl.ds(NCH1, NCH0 - NCH1)])

    plsc.subcore_barrier()

    def start_gather(j, b):
        pltpu.async_copy(p_hbm.at[srcs_v.at[j]], rows[b], gsem[b])

    def wait_gather(b):
        pltpu.make_async_copy(p_hbm.at[srcs_v.at[0]], rows[b], gsem[b]).wait()

    def start_scatter(j, b):
        pltpu.async_copy(rows[b], s_sp.at[dsts_v.at[j]], ssem[b], add=True)

    def wait_scatter(b):
        pltpu.make_async_copy(rows[b], s_sp.at[dsts_v.at[0]], ssem[b]).wait()

    start_gather(0, 0)
    start_gather(1, 1)
    for j in (0, 1):
        wait_gather(j % NBUF)
        start_scatter(j, j % NBUF)
        start_gather(j + LOOKAHEAD, (j + LOOKAHEAD) % NBUF)

    def main_body(g, carry):
        for b in range(NBUF):
            j = g * NBUF + b + LOOKAHEAD
            bb = (b + LOOKAHEAD) % NBUF
            wait_gather(bb)
            start_scatter(j, bb)
            wait_scatter(b)
            start_gather(j + LOOKAHEAD, b)
        return carry

    lax.fori_loop(0, (nch - 2 * LOOKAHEAD) // NBUF, main_body, 0)

    for k in range(LOOKAHEAD, 0, -1):
        j = nch - k
        b = (NBUF - k) % NBUF
        wait_gather(b)
        start_scatter(j, b)
    for b in range(NBUF):
        wait_scatter(b)
    plsc.subcore_barrier()

    pltpu.sync_copy(s_sp.at[stripe], s_out.at[c, stripe])


def _make_sc(width):
    scratch = [
        pltpu.VMEM((NCHMAX, CH), jnp.int32),
        pltpu.VMEM((NCHMAX, CH), jnp.int32),
    ]
    scratch += [pltpu.VMEM((CH, width), _f32)] * NBUF
    scratch += [
        pltpu.VMEM((ZROWS, width), _f32),
        pltpu.VMEM_SHARED((NSINK, width), _f32),
    ]
    scratch += [pltpu.SemaphoreType.DMA] * (2 * NBUF)
    mesh = plsc.VectorSubcoreMesh(core_axis_name="c", subcore_axis_name="s")
    return pl.kernel(
        _sc_body,
        out_type=jax.ShapeDtypeStruct((NSC, NSINK, width), _f32),
        mesh=mesh,
        scratch_types=scratch,
        compiler_params=pltpu.CompilerParams(use_tc_tiling_on_sc=False,
                                             skip_device_barrier=True),
    )


_BN = 1000


def _elu(h):
    return jnp.where(h > 0, h, jnp.exp(jnp.minimum(h, 0.0)) - 1.0)


def _dense_pre_body(x_ref, wl_ref, wr_ref, bl_ref, p_ref, r_ref):
    xb = x_ref[...]
    pj = jnp.dot(xb, wl_ref[...], preferred_element_type=_f32)
    p_ref[...] = jnp.concatenate(
        [pj, jnp.ones((pj.shape[0], 1), _f32), jnp.zeros((pj.shape[0], 15), _f32)],
        axis=1)
    r_ref[...] = jnp.dot(xb, wr_ref[...], preferred_element_type=_f32) + bl_ref[...]


def _dense_pre(x, Wl, Wr, bl):
    n, d = x.shape
    return pl.pallas_call(
        _dense_pre_body,
        grid=(n // _BN,),
        in_specs=[
            pl.BlockSpec((_BN, d), lambda i: (i, 0)),
            pl.BlockSpec((d, DH), lambda i: (0, 0)),
            pl.BlockSpec((d, DH), lambda i: (0, 0)),
            pl.BlockSpec((1, DH), lambda i: (0, 0)),
        ],
        out_specs=[pl.BlockSpec((_BN, 2 * DH), lambda i: (i, 0)),
                   pl.BlockSpec((_BN, DH), lambda i: (i, 0))],
        out_shape=[jax.ShapeDtypeStruct((n, 2 * DH), _f32),
                   jax.ShapeDtypeStruct((n, DH), _f32)],
    )(x, Wl, Wr, bl.reshape(1, DH))


def _combine(s_ref, cnt_ref, r_ref):
    ssum = s_ref[0, :, :DH] + s_ref[1, :, :DH]
    cnt = cnt_ref[0, :, DH:DH + 1] + cnt_ref[1, :, DH:DH + 1]
    inv = 1.0 / jnp.maximum(cnt, 1.0)
    return _elu(ssum * inv + r_ref[...])


def _combine_pre_body(s_ref, cnt_ref, r_ref, wl_ref, wr_ref, bl_ref,
                      p_ref, rout_ref):
    h = _combine(s_ref, cnt_ref, r_ref)
    p_ref[...] = jnp.dot(h, wl_ref[...], preferred_element_type=_f32)
    rout_ref[...] = jnp.dot(h, wr_ref[...], preferred_element_type=_f32) + bl_ref[...]


def _combine_pre(s, cnt, r, Wl, Wr, bl):
    sw = s.shape[-1]
    return pl.pallas_call(
        _combine_pre_body,
        grid=(N // _BN,),
        in_specs=[
            pl.BlockSpec((NSC, _BN, sw), lambda i: (0, i, 0)),
            pl.BlockSpec((NSC, _BN, 2 * DH), lambda i: (0, i, 0)),
            pl.BlockSpec((_BN, DH), lambda i: (i, 0)),
            pl.BlockSpec((DH, DH), lambda i: (0, 0)),
            pl.BlockSpec((DH, DH), lambda i: (0, 0)),
            pl.BlockSpec((1, DH), lambda i: (0, 0)),
        ],
        out_specs=[pl.BlockSpec((_BN, DH), lambda i: (i, 0))] * 2,
        out_shape=[jax.ShapeDtypeStruct((N, DH), _f32)] * 2,
    )(s, cnt, r, Wl, Wr, bl.reshape(1, DH))


def _combine_mlp_body(s_ref, cnt_ref, r_ref, w0, b0, w1, b1, w2, b2, w3, b3,
                      out_ref):
    h = _combine(s_ref, cnt_ref, r_ref)
    h = _elu(jnp.dot(h, w0[...], preferred_element_type=_f32) + b0[...])
    h = _elu(jnp.dot(h, w1[...], preferred_element_type=_f32) + b1[...])
    h = _elu(jnp.dot(h, w2[...], preferred_element_type=_f32) + b2[...])
    out_ref[...] = jnp.dot(h, w3[...], preferred_element_type=_f32) + b3[...]


def _combine_mlp(s, cnt, r, lws):
    (w0, b0), (w1, b1), (w2, b2), (w3, b3) = lws
    d_out = w3.shape[1]
    wspecs = []
    for w, b in lws:
        wspecs.append(pl.BlockSpec(w.shape, lambda i: (0, 0)))
        wspecs.append(pl.BlockSpec((1, b.shape[0]), lambda i: (0, 0)))
    return pl.pallas_call(
        _combine_mlp_body,
        grid=(N // _BN,),
        in_specs=[
            pl.BlockSpec((NSC, _BN, DH), lambda i: (0, i, 0)),
            pl.BlockSpec((NSC, _BN, 2 * DH), lambda i: (0, i, 0)),
            pl.BlockSpec((_BN, DH), lambda i: (i, 0)),
        ] + wspecs,
        out_specs=pl.BlockSpec((_BN, d_out), lambda i: (i, 0)),
        out_shape=jax.ShapeDtypeStruct((N, d_out), _f32),
    )(s, cnt, r, w0, b0.reshape(1, -1), w1, b1.reshape(1, -1),
      w2, b2.reshape(1, -1), w3, b3.reshape(1, -1))


def kernel(x, edge_index,
           conv0_Wl, conv0_bl, conv0_Wr,
           conv1_Wl, conv1_bl, conv1_Wr,
           conv2_Wl, conv2_bl, conv2_Wr,
           lin0_W, lin0_b, lin1_W, lin1_b, lin2_W, lin2_b, lin3_W, lin3_b):
    src = edge_index[0]
    dst = edge_index[1]
    e = src.shape[0]
    pad = EPAD - e
    srcs = jnp.concatenate([src, jnp.zeros((pad,), jnp.int32)]).reshape(NROWS, CH)
    dsts = jnp.concatenate([dst, jnp.full((pad,), N, jnp.int32)]).reshape(NROWS, CH)
    sc32 = _make_sc(2 * DH)
    sc16 = _make_sc(DH)

    p0, r0 = _dense_pre(x, conv0_Wl, conv0_Wr, conv0_bl)
    s0p = sc32(p0, srcs, dsts)
    p1, r1 = _combine_pre(s0p, s0p, r0, conv1_Wl, conv1_Wr, conv1_bl)
    s1p = sc16(p1, srcs, dsts)
    p2, r2 = _combine_pre(s1p, s0p, r1, conv2_Wl, conv2_Wr, conv2_bl)
    s2p = sc16(p2, srcs, dsts)
    return _combine_mlp(s2p, s0p, r2,
                        [(lin0_W, lin0_b), (lin1_W, lin1_b),
                         (lin2_W, lin2_b), (lin3_W, lin3_b)])


# speedup vs baseline: 1.0510x; 1.0487x over previous
import functools

import jax
import jax.numpy as jnp
from jax import lax
from jax.experimental import pallas as pl
from jax.experimental.pallas import tpu as pltpu
from jax.experimental.pallas import tpu_sc as plsc

N = 10000
DH = 16
NSC = 2
NTILES = 16
NW = NSC * NTILES
CH = 256
NCH0 = 72
NCH1 = 8
NCHMAX = max(NCH0, NCH1)
NROWS = NTILES * (NCH0 + NCH1) + NCHMAX
EPAD = NROWS * CH
NSINK = N + 112
ZROWS = NSINK // NTILES
NBUF = 4
LOOKAHEAD = 2

_f32 = jnp.float32


def _sc_body(*refs):
    (p_hbm, srcs_hbm, dsts_hbm,
     s_out,
     srcs_v, dsts_v,
     rb0, rb1, rb2, rb3, zbuf, s_sp,
     gs0, gs1, gs2, gs3, ss0, ss1, ss2, ss3) = refs
    rows = (rb0, rb1, rb2, rb3)
    gsem = (gs0, gs1, gs2, gs3)
    ssem = (ss0, ss1, ss2, ss3)

    c = lax.axis_index("c")
    t = lax.axis_index("s")
    nch = jnp.where(c == 0, NCH0, NCH1)
    row0 = jnp.where(c == 0, t * NCH0, NTILES * NCH0 + t * NCH1)

    stripe = pl.ds(t * ZROWS, ZROWS)
    width = zbuf.shape[1]
    zv = jnp.zeros((16,), _f32)

    def zrow(i, carry):
        for h in range(width // 16):
            zbuf[i, pl.ds(h * 16, 16)] = zv
        return carry

    lax.fori_loop(0, ZROWS, zrow, 0)
    pltpu.sync_copy(zbuf, s_sp.at[stripe])
    pltpu.sync_copy(srcs_hbm.at[pl.ds(row0, NCH1)], srcs_v.at[pl.ds(0, NCH1)])
    pltpu.sync_copy(dsts_hbm.at[pl.ds(row0, NCH1)], dsts_v.at[pl.ds(0, NCH1)])

    @pl.when(c == 0)
    def _copy_extra():
        extra = pl.ds(row0 + NCH1, NCH0 - NCH1)
        pltpu.sync_copy(srcs_hbm.at[extra], srcs_v.at[pl.ds(NCH1, NCH0 - NCH1)])
        pltpu.sync_copy(dsts_hbm.at[extra], dsts_v.at[pl.ds(NCH1, NCH0 - NCH1)])

    plsc.subcore_barrier()

    def start_gather(j, b):
        pltpu.async_copy(p_hbm.at[srcs_v.at[j]], rows[b], gsem[b])

    def wait_gather(b):
        pltpu.make_async_copy(p_hbm.at[srcs_v.at[0]], rows[b], gsem[b]).wait()

    def start_scatter(j, b):
        pltpu.async_copy(rows[b], s_sp.at[dsts_v.at[j]], ssem[b], add=True)

    def wait_scatter(b):
        pltpu.make_async_copy(rows[b], s_sp.at[dsts_v.at[0]], ssem[b]).wait()

    start_gather(0, 0)
    start_gather(1, 1)
    for j in (0, 1):
        wait_gather(j % NBUF)
        start_scatter(j, j % NBUF)
        start_gather(j + LOOKAHEAD, (j + LOOKAHEAD) % NBUF)

    def main_body(g, carry):
        for b in range(NBUF):
            j = g * NBUF + b + LOOKAHEAD
            bb = (b + LOOKAHEAD) % NBUF
            wait_gather(bb)
            start_scatter(j, bb)
            wait_scatter(b)
            start_gather(j + LOOKAHEAD, b)
        return carry

    lax.fori_loop(0, (nch - 2 * LOOKAHEAD) // NBUF, main_body, 0)

    for k in range(LOOKAHEAD, 0, -1):
        j = nch - k
        b = (NBUF - k) % NBUF
        wait_gather(b)
        start_scatter(j, b)
    for b in range(NBUF):
        wait_scatter(b)
    plsc.subcore_barrier()

    pltpu.sync_copy(s_sp.at[stripe], s_out.at[c, stripe])


def _make_sc(width):
    scratch = [
        pltpu.VMEM((NCHMAX, CH), jnp.int32),
        pltpu.VMEM((NCHMAX, CH), jnp.int32),
    ]
    scratch += [pltpu.VMEM((CH, width), _f32)] * NBUF
    scratch += [
        pltpu.VMEM((ZROWS, width), _f32),
        pltpu.VMEM_SHARED((NSINK, width), _f32),
    ]
    scratch += [pltpu.SemaphoreType.DMA] * (2 * NBUF)
    mesh = plsc.VectorSubcoreMesh(core_axis_name="c", subcore_axis_name="s")
    return pl.kernel(
        _sc_body,
        out_type=jax.ShapeDtypeStruct((NSC, NSINK, width), _f32),
        mesh=mesh,
        scratch_types=scratch,
        compiler_params=pltpu.CompilerParams(use_tc_tiling_on_sc=False,
                                             skip_device_barrier=True),
    )


_BN = 1000


def _elu(h):
    return jnp.where(h > 0, h, jnp.exp(jnp.minimum(h, 0.0)) - 1.0)


def _dense_pre_body(x_ref, wl_ref, wr_ref, bl_ref, p_ref, r_ref):
    xb = x_ref[...]
    pj = jnp.dot(xb, wl_ref[...], preferred_element_type=_f32)
    p_ref[...] = jnp.concatenate(
        [pj, jnp.ones((pj.shape[0], 1), _f32), jnp.zeros((pj.shape[0], 15), _f32)],
        axis=1)
    r_ref[...] = jnp.dot(xb, wr_ref[...], preferred_element_type=_f32) + bl_ref[...]


def _dense_pre(x, Wl, Wr, bl):
    n, d = x.shape
    return pl.pallas_call(
        _dense_pre_body,
        grid=(n // _BN,),
        in_specs=[
            pl.BlockSpec((_BN, d), lambda i: (i, 0)),
            pl.BlockSpec((d, DH), lambda i: (0, 0)),
            pl.BlockSpec((d, DH), lambda i: (0, 0)),
            pl.BlockSpec((1, DH), lambda i: (0, 0)),
        ],
        out_specs=[pl.BlockSpec((_BN, 2 * DH), lambda i: (i, 0)),
                   pl.BlockSpec((_BN, DH), lambda i: (i, 0))],
        out_shape=[jax.ShapeDtypeStruct((n, 2 * DH), _f32),
                   jax.ShapeDtypeStruct((n, DH), _f32)],
    )(x, Wl, Wr, bl.reshape(1, DH))


def _combine(s_ref, cnt_ref, r_ref):
    ssum = s_ref[0, :, :DH] + s_ref[1, :, :DH]
    cnt = cnt_ref[0, :, DH:DH + 1] + cnt_ref[1, :, DH:DH + 1]
    inv = 1.0 / jnp.maximum(cnt, 1.0)
    return _elu(ssum * inv + r_ref[...])


def _combine_pre_body(s_ref, cnt_ref, r_ref, wl_ref, wr_ref, bl_ref,
                      p_ref, rout_ref):
    h = _combine(s_ref, cnt_ref, r_ref)
    p_ref[...] = jnp.dot(h, wl_ref[...], preferred_element_type=_f32)
    rout_ref[...] = jnp.dot(h, wr_ref[...], preferred_element_type=_f32) + bl_ref[...]


def _combine_pre(s, cnt, r, Wl, Wr, bl):
    sw = s.shape[-1]
    return pl.pallas_call(
        _combine_pre_body,
        grid=(N // _BN,),
        in_specs=[
            pl.BlockSpec((NSC, _BN, sw), lambda i: (0, i, 0)),
            pl.BlockSpec((NSC, _BN, 2 * DH), lambda i: (0, i, 0)),
            pl.BlockSpec((_BN, DH), lambda i: (i, 0)),
            pl.BlockSpec((DH, DH), lambda i: (0, 0)),
            pl.BlockSpec((DH, DH), lambda i: (0, 0)),
            pl.BlockSpec((1, DH), lambda i: (0, 0)),
        ],
        out_specs=[pl.BlockSpec((_BN, DH), lambda i: (i, 0))] * 2,
        out_shape=[jax.ShapeDtypeStruct((N, DH), _f32)] * 2,
    )(s, cnt, r, Wl, Wr, bl.reshape(1, DH))


def _combine_mlp_body(s_ref, cnt_ref, r_ref, w0, b0, w1, b1, w2, b2, w3, b3,
                      out_ref):
    h = _combine(s_ref, cnt_ref, r_ref)
    h = _elu(jnp.dot(h, w0[...], preferred_element_type=_f32) + b0[...])
    h = _elu(jnp.dot(h, w1[...], preferred_element_type=_f32) + b1[...])
    h = _elu(jnp.dot(h, w2[...], preferred_element_type=_f32) + b2[...])
    out_ref[...] = jnp.dot(h, w3[...], preferred_element_type=_f32) + b3[...]


def _combine_mlp(s, cnt, r, lws):
    (w0, b0), (w1, b1), (w2, b2), (w3, b3) = lws
    d_out = w3.shape[1]
    wspecs = []
    for w, b in lws:
        wspecs.append(pl.BlockSpec(w.shape, lambda i: (0, 0)))
        wspecs.append(pl.BlockSpec((1, b.shape[0]), lambda i: (0, 0)))
    return pl.pallas_call(
        _combine_mlp_body,
        grid=(N // _BN,),
        in_specs=[
            pl.BlockSpec((NSC, _BN, DH), lambda i: (0, i, 0)),
            pl.BlockSpec((NSC, _BN, 2 * DH), lambda i: (0, i, 0)),
            pl.BlockSpec((_BN, DH), lambda i: (i, 0)),
        ] + wspecs,
        out_specs=pl.BlockSpec((_BN, d_out), lambda i: (i, 0)),
        out_shape=jax.ShapeDtypeStruct((N, d_out), _f32),
    )(s, cnt, r, w0, b0.reshape(1, -1), w1, b1.reshape(1, -1),
      w2, b2.reshape(1, -1), w3, b3.reshape(1, -1))


def kernel(x, edge_index,
           conv0_Wl, conv0_bl, conv0_Wr,
           conv1_Wl, conv1_bl, conv1_Wr,
           conv2_Wl, conv2_bl, conv2_Wr,
           lin0_W, lin0_b, lin1_W, lin1_b, lin2_W, lin2_b, lin3_W, lin3_b):
    src = edge_index[0]
    dst = edge_index[1]
    e = src.shape[0]
    pad = EPAD - e
    srcs = jnp.concatenate([src, jnp.zeros((pad,), jnp.int32)]).reshape(NROWS, CH)
    dsts = jnp.concatenate([dst, jnp.full((pad,), N, jnp.int32)]).reshape(NROWS, CH)
    sc32 = _make_sc(2 * DH)
    sc16 = _make_sc(DH)

    p0, r0 = _dense_pre(x, conv0_Wl, conv0_Wr, conv0_bl)
    s0p = sc32(p0, srcs, dsts)
    p1, r1 = _combine_pre(s0p, s0p, r0, conv1_Wl, conv1_Wr, conv1_bl)
    s1p = sc16(p1, srcs, dsts)
    p2, r2 = _combine_pre(s1p, s0p, r1, conv2_Wl, conv2_Wr, conv2_bl)
    s2p = sc16(p2, srcs, dsts)
    return _combine_mlp(s2p, s0p, r2,
                        [(lin0_W, lin0_b), (lin1_W, lin1_b),
                         (lin2_W, lin2_b), (lin3_W, lin3_b)])
